# baseline (speedup 1.0000x reference)
import functools

import jax
import jax.numpy as jnp
from jax import lax
from jax.experimental import pallas as pl
from jax.experimental.pallas import tpu as pltpu

D_MODEL = 1024
D_SAE = 16384
N_TOK = 8192
TOPK = 32


BT_ENC = 512
BD_ENC = 2048


def _enc_body(x_ref, w_ref, b_ref, out_ref):
    acc = jnp.dot(x_ref[...], w_ref[...], preferred_element_type=jnp.float32)
    out_ref[...] = acc + b_ref[...]


def _encoder(x, W_enc, b_enc):
    grid = (D_SAE // BD_ENC, N_TOK // BT_ENC)
    return pl.pallas_call(
        _enc_body,
        grid=grid,
        in_specs=[
            pl.BlockSpec((BT_ENC, D_MODEL), lambda j, i: (i, 0)),
            pl.BlockSpec((D_MODEL, BD_ENC), lambda j, i: (0, j)),
            pl.BlockSpec((1, BD_ENC), lambda j, i: (0, j)),
        ],
        out_specs=pl.BlockSpec((BT_ENC, BD_ENC), lambda j, i: (i, j)),
        out_shape=jax.ShapeDtypeStruct((N_TOK, D_SAE), jnp.float32),
    )(x, W_enc, b_enc.reshape(1, D_SAE))


BT_DEC = 256


def _dec_body(s_ref, w_ref, b_ref, out_ref):
    acc = jnp.dot(s_ref[...], w_ref[...], preferred_element_type=jnp.float32)
    out_ref[...] = acc + b_ref[...]


def _decoder(sparse_bf16, W_dec_bf16, b_dec):
    grid = (N_TOK // BT_DEC,)
    return pl.pallas_call(
        _dec_body,
        grid=grid,
        in_specs=[
            pl.BlockSpec((BT_DEC, D_SAE), lambda i: (i, 0)),
            pl.BlockSpec((D_SAE, D_MODEL), lambda i: (0, 0)),
            pl.BlockSpec((1, D_MODEL), lambda i: (0, 0)),
        ],
        out_specs=pl.BlockSpec((BT_DEC, D_MODEL), lambda i: (i, 0)),
        out_shape=jax.ShapeDtypeStruct((N_TOK, D_MODEL), jnp.float32),
    )(sparse_bf16, W_dec_bf16, b_dec.reshape(1, D_MODEL))


def kernel(x, W_enc, b_enc, W_dec, b_dec):
    latents = _encoder(x, W_enc, b_enc)
    topk_values, topk_indices = jax.lax.top_k(latents, TOPK)
    rows = jnp.arange(N_TOK)[:, None]
    sparse_latents = jnp.zeros_like(latents).at[rows, topk_indices].set(topk_values)
    recon = _decoder(sparse_latents.astype(jnp.bfloat16),
                     W_dec.astype(jnp.bfloat16), b_dec)
    return (recon, sparse_latents, latents)


# speedup vs baseline: 4.6448x; 4.6448x over previous
import functools

import jax
import jax.numpy as jnp
from jax import lax
from jax.experimental import pallas as pl
from jax.experimental.pallas import tpu as pltpu
from jax.experimental.pallas import tpu_sc as plsc

D_MODEL = 1024
D_SAE = 16384
N_TOK = 8192
TOPK = 32

NEG_INF = float("-inf")
L = 16
N_WORKERS = 32
ROWS_PER_W = N_TOK // N_WORKERS
N_GROUPS = 64
GROUP_VREGS = D_SAE // N_GROUPS // L
CAP_VREGS = 256


BT_ENC = 512
BD_ENC = 2048


def _enc_body(x_ref, w_ref, b_ref, out_ref):
    acc = jnp.dot(x_ref[...], w_ref[...], preferred_element_type=jnp.float32)
    out_ref[...] = acc + b_ref[...]


def _encoder(x, W_enc, b_enc):
    grid = (D_SAE // BD_ENC, N_TOK // BT_ENC)
    return pl.pallas_call(
        _enc_body,
        grid=grid,
        in_specs=[
            pl.BlockSpec((BT_ENC, D_MODEL), lambda j, i: (i, 0)),
            pl.BlockSpec((D_MODEL, BD_ENC), lambda j, i: (0, j)),
            pl.BlockSpec((1, BD_ENC), lambda j, i: (0, j)),
        ],
        out_specs=pl.BlockSpec((BT_ENC, BD_ENC), lambda j, i: (i, j)),
        out_shape=jax.ShapeDtypeStruct((N_TOK, D_SAE), jnp.float32),
    )(x, W_enc, b_enc.reshape(1, D_SAE))


BT_DEC = 128


def _dec_body(s_ref, w_ref, b_ref, out_ref):
    s = s_ref[...].astype(jnp.bfloat16)
    acc = jnp.dot(s, w_ref[...], preferred_element_type=jnp.float32)
    out_ref[...] = acc + b_ref[...]


def _decoder(sparse, W_dec_bf16, b_dec):
    grid = (N_TOK // BT_DEC,)
    return pl.pallas_call(
        _dec_body,
        grid=grid,
        in_specs=[
            pl.BlockSpec((BT_DEC, D_SAE), lambda i: (i, 0)),
            pl.BlockSpec((D_SAE, D_MODEL), lambda i: (0, 0)),
            pl.BlockSpec((1, D_MODEL), lambda i: (0, 0)),
        ],
        out_specs=pl.BlockSpec((BT_DEC, D_MODEL), lambda i: (i, 0)),
        out_shape=jax.ShapeDtypeStruct((N_TOK, D_MODEL), jnp.float32),
    )(sparse, W_dec_bf16, b_dec.reshape(1, D_MODEL))


def _any(mask):
    return plsc.all_reduce_population_count(mask)[0] > 0


def _sort_desc(v):
    s, _ = plsc.sort_key_val(v, v, descending=True)
    return s


def _merge16_vals(t_hi, t_lo, s_desc):
    x = jnp.maximum(t_lo, lax.rev(s_desc, (0,)))
    xs = _sort_desc(x)
    rxs = lax.rev(xs, (0,))
    hi = jnp.maximum(t_hi, rxs)
    lo = jnp.minimum(t_hi, rxs)
    return _sort_desc(hi), _sort_desc(lo)


def _sort_kv_desc(v, i):
    return plsc.sort_key_val(v, i, descending=True)


def _merge16_kv(tv_hi, ti_hi, tv_lo, ti_lo, sv_desc, si_desc):
    rv = lax.rev(sv_desc, (0,))
    ri = lax.rev(si_desc, (0,))
    c = tv_lo >= rv
    xv = jnp.where(c, tv_lo, rv)
    xi = jnp.where(c, ti_lo, ri)
    xv, xi = _sort_kv_desc(xv, xi)
    rxv = lax.rev(xv, (0,))
    rxi = lax.rev(xi, (0,))
    c2 = tv_hi >= rxv
    hv = jnp.where(c2, tv_hi, rxv)
    hi_ = jnp.where(c2, ti_hi, rxi)
    lv = jnp.where(c2, rxv, tv_hi)
    li = jnp.where(c2, rxi, ti_hi)
    tv_hi, ti_hi = _sort_kv_desc(hv, hi_)
    tv_lo, ti_lo = _sort_kv_desc(lv, li)
    return tv_hi, ti_hi, tv_lo, ti_lo


def _row_topk_scatter(buf, qmax_ref, cand_v, cand_i, out_ref, prev_i,
                      scal_ref):
    lane = jnp.arange(L, dtype=jnp.int32)
    zero16 = jnp.zeros((L,), jnp.float32)

    def a_body(g, _):
        base = g * (GROUP_VREGS * L)
        m = buf[pl.ds(base, L)]
        for i in range(1, GROUP_VREGS):
            m = jnp.maximum(m, buf[pl.ds(base + i * L, L)])
        qmax_ref[pl.ds(g * L, L)] = m
        return 0
    lax.fori_loop(0, N_GROUPS, a_body, 0)

    def b_body(g, carry):
        t_hi, t_lo, tmin = carry
        qm = qmax_ref[pl.ds(g * L, L)]

        def do(c):
            th, tl, _ = c
            th, tl = _merge16_vals(th, tl, _sort_desc(qm))
            return th, tl, tl[15]

        return lax.cond(_any(qm > tmin), do, lambda c: c,
                        (t_hi, t_lo, tmin))
    _, _, t1 = lax.fori_loop(
        0, N_GROUPS, b_body,
        (jnp.full((L,), NEG_INF, jnp.float32),
         jnp.full((L,), NEG_INF, jnp.float32), jnp.float32(NEG_INF)))

    def c_body(g, off):
        qm = qmax_ref[pl.ds(g * L, L)]

        def do_group(off):
            def v_body(i, off):
                base = g * (GROUP_VREGS * L) + i * L
                v = buf[pl.ds(base, L)]
                m = v >= t1

                def store(off):
                    cand_v[pl.ds(off * L, L)] = jnp.where(m, v, NEG_INF)
                    cand_i[pl.ds(off * L, L)] = lane + base
                    return off + 1

                return lax.cond(_any(m) & (off < CAP_VREGS), store,
                                lambda o: o, off)
            return lax.fori_loop(0, GROUP_VREGS, v_body, off)

        return lax.cond(_any(qm >= t1), do_group, lambda o: o, off)
    n_cand = lax.fori_loop(0, N_GROUPS, c_body, jnp.int32(0))

    def d_body(j, carry):
        tv_hi, ti_hi, tv_lo, ti_lo, tmin = carry
        v = cand_v[pl.ds(j * L, L)]
        i = cand_i[pl.ds(j * L, L)]

        def do(c):
            vh, ih, vl, il, _ = c
            sv, si = _sort_kv_desc(v, i)
            vh, ih, vl, il = _merge16_kv(vh, ih, vl, il, sv, si)
            return vh, ih, vl, il, vl[15]

        return lax.cond(_any(v > tmin), do, lambda c: c, carry)
    tv_hi, ti_hi, tv_lo, ti_lo, _ = lax.fori_loop(
        0, n_cand, d_body,
        (jnp.full((L,), NEG_INF, jnp.float32), jnp.zeros((L,), jnp.int32),
         jnp.full((L,), NEG_INF, jnp.float32), jnp.zeros((L,), jnp.int32),
         jnp.float32(NEG_INF)))

    plsc.store_scatter(out_ref, [prev_i[pl.ds(0, L)]], zero16)
    plsc.store_scatter(out_ref, [prev_i[pl.ds(L, L)]], zero16)
    plsc.store_scatter(out_ref, [ti_hi], tv_hi)
    plsc.store_scatter(out_ref, [ti_lo], tv_lo)
    prev_i[pl.ds(0, L)] = ti_hi
    prev_i[pl.ds(L, L)] = ti_lo


def _sc_topk_scatter(latents):
    mesh = plsc.VectorSubcoreMesh(core_axis_name="c", subcore_axis_name="s")

    @functools.partial(
        pl.kernel,
        out_type=jax.ShapeDtypeStruct((N_TOK, D_SAE), jnp.float32),
        mesh=mesh,
        compiler_params=pltpu.CompilerParams(needs_layout_passes=False),
        scratch_types=[
            pltpu.VMEM((D_SAE,), jnp.float32),
            pltpu.VMEM((D_SAE,), jnp.float32),
            pltpu.VMEM((D_SAE,), jnp.float32),
            pltpu.VMEM((N_GROUPS * L,), jnp.float32),
            pltpu.VMEM((CAP_VREGS * L,), jnp.float32),
            pltpu.VMEM((CAP_VREGS * L,), jnp.int32),
            pltpu.VMEM((2 * L,), jnp.int32),
            pltpu.VMEM((L,), jnp.float32),
            pltpu.SemaphoreType.DMA,
            pltpu.SemaphoreType.DMA,
            pltpu.SemaphoreType.DMA,
        ],
    )
    def sc_kernel(lat_hbm, out_hbm, inbuf0, inbuf1, outbuf, qmax_ref, cand_v,
                  cand_i, prev_i, scal_ref, sem0, sem1, osem):
        wid = lax.axis_index("s") * 2 + lax.axis_index("c")
        row0 = wid * ROWS_PER_W
        lane = jnp.arange(L, dtype=jnp.int32)

        def z_body(i, _):
            outbuf[pl.ds(i * L, L)] = jnp.zeros((L,), jnp.float32)
            return 0
        lax.fori_loop(0, D_SAE // L, z_body, 0)
        prev_i[pl.ds(0, L)] = lane
        prev_i[pl.ds(L, L)] = lane + L

        in_sems = (sem0, sem1)
        in_bufs = (inbuf0, inbuf1)
        pltpu.async_copy(lat_hbm.at[row0], inbuf0, sem0)

        def pair_body(p, _):
            for phase in range(2):
                r = p * 2 + phase
                b = phase
                @pl.when(r + 1 < ROWS_PER_W)
                def _():
                    pltpu.async_copy(lat_hbm.at[row0 + r + 1],
                                     in_bufs[1 - b], in_sems[1 - b])
                pltpu.make_async_copy(lat_hbm.at[row0 + r], in_bufs[b],
                                      in_sems[b]).wait()
                @pl.when(r > 0)
                def _():
                    pltpu.make_async_copy(outbuf, out_hbm.at[row0 + r - 1],
                                          osem).wait()
                _row_topk_scatter(in_bufs[b], qmax_ref, cand_v, cand_i,
                                  outbuf, prev_i, scal_ref)
                pltpu.async_copy(outbuf, out_hbm.at[row0 + r], osem)
            return 0
        lax.fori_loop(0, ROWS_PER_W // 2, pair_body, 0)
        pltpu.make_async_copy(outbuf, out_hbm.at[row0 + ROWS_PER_W - 1],
                              osem).wait()

    return sc_kernel(latents)


def kernel(x, W_enc, b_enc, W_dec, b_dec):
    latents = _encoder(x, W_enc, b_enc)
    sparse_latents = _sc_topk_scatter(latents)
    recon = _decoder(sparse_latents, W_dec.astype(jnp.bfloat16), b_dec)
    return (recon, sparse_latents, latents)


# speedup vs baseline: 6.4264x; 1.3836x over previous
import functools

import jax
import jax.numpy as jnp
from jax import lax
from jax.experimental import pallas as pl
from jax.experimental.pallas import tpu as pltpu
from jax.experimental.pallas import tpu_sc as plsc

D_MODEL = 1024
D_SAE = 16384
N_TOK = 8192
TOPK = 32

NEG_INF = float("-inf")
L = 16
N_WORKERS = 32
ROWS_PER_W = N_TOK // N_WORKERS
N_GROUPS = 64
GROUP_VREGS = D_SAE // N_GROUPS // L
CAP_ELEMS = 2048


BT_ENC = 512
BD_ENC = 2048


def _enc_body(x_ref, w_ref, b_ref, out_ref):
    acc = jnp.dot(x_ref[...], w_ref[...], preferred_element_type=jnp.float32)
    out_ref[...] = acc + b_ref[...]


def _encoder(x, W_enc, b_enc):
    grid = (D_SAE // BD_ENC, N_TOK // BT_ENC)
    return pl.pallas_call(
        _enc_body,
        grid=grid,
        in_specs=[
            pl.BlockSpec((BT_ENC, D_MODEL), lambda j, i: (i, 0)),
            pl.BlockSpec((D_MODEL, BD_ENC), lambda j, i: (0, j)),
            pl.BlockSpec((1, BD_ENC), lambda j, i: (0, j)),
        ],
        out_specs=pl.BlockSpec((BT_ENC, BD_ENC), lambda j, i: (i, j)),
        out_shape=jax.ShapeDtypeStruct((N_TOK, D_SAE), jnp.float32),
    )(x, W_enc, b_enc.reshape(1, D_SAE))


BT_DEC = 128


def _dec_body(s_ref, w_ref, b_ref, out_ref):
    s = s_ref[...].astype(jnp.bfloat16)
    acc = jnp.dot(s, w_ref[...], preferred_element_type=jnp.float32)
    out_ref[...] = acc + b_ref[...]


def _decoder(sparse, W_dec_bf16, b_dec):
    grid = (N_TOK // BT_DEC,)
    return pl.pallas_call(
        _dec_body,
        grid=grid,
        in_specs=[
            pl.BlockSpec((BT_DEC, D_SAE), lambda i: (i, 0)),
            pl.BlockSpec((D_SAE, D_MODEL), lambda i: (0, 0)),
            pl.BlockSpec((1, D_MODEL), lambda i: (0, 0)),
        ],
        out_specs=pl.BlockSpec((BT_DEC, D_MODEL), lambda i: (i, 0)),
        out_shape=jax.ShapeDtypeStruct((N_TOK, D_MODEL), jnp.float32),
    )(sparse, W_dec_bf16, b_dec.reshape(1, D_MODEL))


def _any(mask):
    return plsc.all_reduce_population_count(mask)[0] > 0


def _sort_desc(v):
    s, _ = plsc.sort_key_val(v, v, descending=True)
    return s


def _merge16_vals(t_hi, t_lo, s_desc):
    x = jnp.maximum(t_lo, lax.rev(s_desc, (0,)))
    xs = _sort_desc(x)
    rxs = lax.rev(xs, (0,))
    hi = jnp.maximum(t_hi, rxs)
    lo = jnp.minimum(t_hi, rxs)
    return _sort_desc(hi), _sort_desc(lo)


def _sort_kv_desc(v, i):
    return plsc.sort_key_val(v, i, descending=True)


def _merge16_kv(tv_hi, ti_hi, tv_lo, ti_lo, sv_desc, si_desc):
    rv = lax.rev(sv_desc, (0,))
    ri = lax.rev(si_desc, (0,))
    c = tv_lo >= rv
    xv = jnp.where(c, tv_lo, rv)
    xi = jnp.where(c, ti_lo, ri)
    xv, xi = _sort_kv_desc(xv, xi)
    rxv = lax.rev(xv, (0,))
    rxi = lax.rev(xi, (0,))
    c2 = tv_hi >= rxv
    hv = jnp.where(c2, tv_hi, rxv)
    hi_ = jnp.where(c2, ti_hi, rxi)
    lv = jnp.where(c2, rxv, tv_hi)
    li = jnp.where(c2, rxi, ti_hi)
    tv_hi, ti_hi = _sort_kv_desc(hv, hi_)
    tv_lo, ti_lo = _sort_kv_desc(lv, li)
    return tv_hi, ti_hi, tv_lo, ti_lo


def _row_topk_scatter(buf, qmax_ref, cand_v, cand_i, out_ref, prev_i,
                      scal_ref):
    lane = jnp.arange(L, dtype=jnp.int32)
    zero16 = jnp.zeros((L,), jnp.float32)

    def a_body(g, _):
        base = g * (GROUP_VREGS * L)
        m = buf[pl.ds(base, L)]
        for i in range(1, GROUP_VREGS):
            m = jnp.maximum(m, buf[pl.ds(base + i * L, L)])
        qmax_ref[pl.ds(g * L, L)] = m
        return 0
    lax.fori_loop(0, N_GROUPS, a_body, 0, unroll=4)

    def b_body(g, carry):
        t_hi, t_lo, tmin = carry
        qm = qmax_ref[pl.ds(g * L, L)]

        def do(c):
            th, tl, _ = c
            th, tl = _merge16_vals(th, tl, _sort_desc(qm))
            return th, tl, tl[15]

        return lax.cond(_any(qm > tmin), do, lambda c: c,
                        (t_hi, t_lo, tmin))
    _, _, t1 = lax.fori_loop(
        0, N_GROUPS, b_body,
        (jnp.full((L,), NEG_INF, jnp.float32),
         jnp.full((L,), NEG_INF, jnp.float32), jnp.float32(NEG_INF)))

    def c_body(g, off):
        qm = qmax_ref[pl.ds(g * L, L)]

        def do_group(off):
            def v_body(i, off):
                base = g * (GROUP_VREGS * L) + i * L
                v = buf[pl.ds(base, L)]
                m = v >= t1
                plsc.store_compressed(cand_v.at[pl.ds(off, L)], v, mask=m)
                plsc.store_compressed(cand_i.at[pl.ds(off, L)], lane + base,
                                      mask=m)
                return off + plsc.all_reduce_population_count(m)[0]
            return lax.fori_loop(0, GROUP_VREGS, v_body, off, unroll=4)

        return lax.cond(_any(qm >= t1) & (off < CAP_ELEMS - GROUP_VREGS * L),
                        do_group, lambda o: o, off)
    n_cand = lax.fori_loop(0, N_GROUPS, c_body, jnp.int32(0))
    cand_v[pl.ds(n_cand, L)] = jnp.full((L,), NEG_INF, jnp.float32)
    n_cvregs = (n_cand + L - 1) // L

    def d_body(j, carry):
        tv_hi, ti_hi, tv_lo, ti_lo, tmin = carry
        v = cand_v[pl.ds(j * L, L)]
        i = cand_i[pl.ds(j * L, L)]

        def do(c):
            vh, ih, vl, il, _ = c
            sv, si = _sort_kv_desc(v, i)
            vh, ih, vl, il = _merge16_kv(vh, ih, vl, il, sv, si)
            return vh, ih, vl, il, vl[15]

        return lax.cond(_any(v > tmin), do, lambda c: c, carry)
    tv_hi, ti_hi, tv_lo, ti_lo, _ = lax.fori_loop(
        0, n_cvregs, d_body,
        (jnp.full((L,), NEG_INF, jnp.float32), jnp.zeros((L,), jnp.int32),
         jnp.full((L,), NEG_INF, jnp.float32), jnp.zeros((L,), jnp.int32),
         jnp.float32(NEG_INF)))

    plsc.store_scatter(out_ref, [prev_i[pl.ds(0, L)]], zero16)
    plsc.store_scatter(out_ref, [prev_i[pl.ds(L, L)]], zero16)
    plsc.store_scatter(out_ref, [ti_hi], tv_hi)
    plsc.store_scatter(out_ref, [ti_lo], tv_lo)
    prev_i[pl.ds(0, L)] = ti_hi
    prev_i[pl.ds(L, L)] = ti_lo


def _sc_topk_scatter(latents):
    mesh = plsc.VectorSubcoreMesh(core_axis_name="c", subcore_axis_name="s")

    @functools.partial(
        pl.kernel,
        out_type=jax.ShapeDtypeStruct((N_TOK, D_SAE), jnp.float32),
        mesh=mesh,
        compiler_params=pltpu.CompilerParams(needs_layout_passes=False),
        scratch_types=[
            pltpu.VMEM((D_SAE,), jnp.float32),
            pltpu.VMEM((D_SAE,), jnp.float32),
            pltpu.VMEM((D_SAE,), jnp.float32),
            pltpu.VMEM((N_GROUPS * L,), jnp.float32),
            pltpu.VMEM((CAP_ELEMS + L,), jnp.float32),
            pltpu.VMEM((CAP_ELEMS + L,), jnp.int32),
            pltpu.VMEM((2 * L,), jnp.int32),
            pltpu.VMEM((L,), jnp.float32),
            pltpu.SemaphoreType.DMA,
            pltpu.SemaphoreType.DMA,
            pltpu.SemaphoreType.DMA,
        ],
    )
    def sc_kernel(lat_hbm, out_hbm, inbuf0, inbuf1, outbuf, qmax_ref, cand_v,
                  cand_i, prev_i, scal_ref, sem0, sem1, osem):
        wid = lax.axis_index("s") * 2 + lax.axis_index("c")
        row0 = wid * ROWS_PER_W
        lane = jnp.arange(L, dtype=jnp.int32)

        def z_body(i, _):
            outbuf[pl.ds(i * L, L)] = jnp.zeros((L,), jnp.float32)
            return 0
        lax.fori_loop(0, D_SAE // L, z_body, 0)
        prev_i[pl.ds(0, L)] = lane
        prev_i[pl.ds(L, L)] = lane + L

        in_sems = (sem0, sem1)
        in_bufs = (inbuf0, inbuf1)
        pltpu.async_copy(lat_hbm.at[row0], inbuf0, sem0)

        def pair_body(p, _):
            for phase in range(2):
                r = p * 2 + phase
                b = phase
                @pl.when(r + 1 < ROWS_PER_W)
                def _():
                    pltpu.async_copy(lat_hbm.at[row0 + r + 1],
                                     in_bufs[1 - b], in_sems[1 - b])
                pltpu.make_async_copy(lat_hbm.at[row0 + r], in_bufs[b],
                                      in_sems[b]).wait()
                @pl.when(r > 0)
                def _():
                    pltpu.make_async_copy(outbuf, out_hbm.at[row0 + r - 1],
                                          osem).wait()
                _row_topk_scatter(in_bufs[b], qmax_ref, cand_v, cand_i,
                                  outbuf, prev_i, scal_ref)
                pltpu.async_copy(outbuf, out_hbm.at[row0 + r], osem)
            return 0
        lax.fori_loop(0, ROWS_PER_W // 2, pair_body, 0)
        pltpu.make_async_copy(outbuf, out_hbm.at[row0 + ROWS_PER_W - 1],
                              osem).wait()

    return sc_kernel(latents)


def kernel(x, W_enc, b_enc, W_dec, b_dec):
    latents = _encoder(x, W_enc, b_enc)
    sparse_latents = _sc_topk_scatter(latents)
    recon = _decoder(sparse_latents, W_dec.astype(jnp.bfloat16), b_dec)
    return (recon, sparse_latents, latents)


# speedup vs baseline: 9.3129x; 1.4492x over previous
import functools

import jax
import jax.numpy as jnp
from jax import lax
from jax.experimental import pallas as pl
from jax.experimental.pallas import tpu as pltpu
from jax.experimental.pallas import tpu_sc as plsc

D_MODEL = 1024
D_SAE = 16384
N_TOK = 8192
TOPK = 32

NEG_INF = float("-inf")
L = 16
N_WORKERS = 32
ROWS_PER_W = N_TOK // N_WORKERS
N_GROUPS = 64
GROUP_VREGS = D_SAE // N_GROUPS // L
CAP_ELEMS = 2048


BT_ENC = 512
BD_ENC = 2048


def _enc_body(x_ref, w_ref, b_ref, out_ref):
    acc = jnp.dot(x_ref[...], w_ref[...], preferred_element_type=jnp.float32)
    out_ref[...] = acc + b_ref[...]


def _encoder(x, W_enc, b_enc):
    grid = (D_SAE // BD_ENC, N_TOK // BT_ENC)
    return pl.pallas_call(
        _enc_body,
        grid=grid,
        in_specs=[
            pl.BlockSpec((BT_ENC, D_MODEL), lambda j, i: (i, 0)),
            pl.BlockSpec((D_MODEL, BD_ENC), lambda j, i: (0, j)),
            pl.BlockSpec((1, BD_ENC), lambda j, i: (0, j)),
        ],
        out_specs=pl.BlockSpec((BT_ENC, BD_ENC), lambda j, i: (i, j)),
        out_shape=jax.ShapeDtypeStruct((N_TOK, D_SAE), jnp.float32),
    )(x, W_enc, b_enc.reshape(1, D_SAE))


BT_DEC = 128


def _dec_body(s_ref, w_ref, b_ref, out_ref):
    s = s_ref[...].astype(jnp.bfloat16)
    acc = jnp.dot(s, w_ref[...], preferred_element_type=jnp.float32)
    out_ref[...] = acc + b_ref[...]


def _decoder(sparse, W_dec_bf16, b_dec):
    grid = (N_TOK // BT_DEC,)
    return pl.pallas_call(
        _dec_body,
        grid=grid,
        in_specs=[
            pl.BlockSpec((BT_DEC, D_SAE), lambda i: (i, 0)),
            pl.BlockSpec((D_SAE, D_MODEL), lambda i: (0, 0)),
            pl.BlockSpec((1, D_MODEL), lambda i: (0, 0)),
        ],
        out_specs=pl.BlockSpec((BT_DEC, D_MODEL), lambda i: (i, 0)),
        out_shape=jax.ShapeDtypeStruct((N_TOK, D_MODEL), jnp.float32),
    )(sparse, W_dec_bf16, b_dec.reshape(1, D_MODEL))


def _any(mask):
    return plsc.all_reduce_population_count(mask)[0] > 0


def _sort_desc(v):
    s, _ = plsc.sort_key_val(v, v, descending=True)
    return s


def _merge16_vals(t_hi, t_lo, s_desc):
    x = jnp.maximum(t_lo, lax.rev(s_desc, (0,)))
    xs = _sort_desc(x)
    rxs = lax.rev(xs, (0,))
    hi = jnp.maximum(t_hi, rxs)
    lo = jnp.minimum(t_hi, rxs)
    return _sort_desc(hi), _sort_desc(lo)


def _sort_kv_desc(v, i):
    return plsc.sort_key_val(v, i, descending=True)


def _merge16_kv(tv_hi, ti_hi, tv_lo, ti_lo, sv_desc, si_desc):
    rv = lax.rev(sv_desc, (0,))
    ri = lax.rev(si_desc, (0,))
    c = tv_lo >= rv
    xv = jnp.where(c, tv_lo, rv)
    xi = jnp.where(c, ti_lo, ri)
    xv, xi = _sort_kv_desc(xv, xi)
    rxv = lax.rev(xv, (0,))
    rxi = lax.rev(xi, (0,))
    c2 = tv_hi >= rxv
    hv = jnp.where(c2, tv_hi, rxv)
    hi_ = jnp.where(c2, ti_hi, rxi)
    lv = jnp.where(c2, rxv, tv_hi)
    li = jnp.where(c2, rxi, ti_hi)
    tv_hi, ti_hi = _sort_kv_desc(hv, hi_)
    tv_lo, ti_lo = _sort_kv_desc(lv, li)
    return tv_hi, ti_hi, tv_lo, ti_lo


def _row_topk_scatter(buf, qmax_ref, sid_ref, out_ref, prev_i):
    lane = jnp.arange(L, dtype=jnp.int32)
    zero16 = jnp.zeros((L,), jnp.float32)

    def a_body(g, _):
        base = g * (GROUP_VREGS * L)
        m = buf[pl.ds(base, L)]
        for i in range(1, GROUP_VREGS):
            m = jnp.maximum(m, buf[pl.ds(base + i * L, L)])
        qmax_ref[pl.ds(g * L, L)] = m
        return 0
    lax.fori_loop(0, N_GROUPS, a_body, 0, unroll=4)

    def b_body(g, carry):
        t_hi, t_lo, tmin = carry
        qm = qmax_ref[pl.ds(g * L, L)]

        def do(c):
            th, tl, _ = c
            th, tl = _merge16_vals(th, tl, _sort_desc(qm))
            return th, tl, tl[15]

        return lax.cond(_any(qm > tmin), do, lambda c: c,
                        (t_hi, t_lo, tmin))
    _, _, t1 = lax.fori_loop(
        0, N_GROUPS, b_body,
        (jnp.full((L,), NEG_INF, jnp.float32),
         jnp.full((L,), NEG_INF, jnp.float32), jnp.float32(NEG_INF)))

    def c_body(g, off):
        qm = qmax_ref[pl.ds(g * L, L)]
        m = qm >= t1
        plsc.store_compressed(sid_ref.at[pl.ds(off, L)], lane + g * L, mask=m)
        return off + plsc.all_reduce_population_count(m)[0]
    lax.fori_loop(0, N_GROUPS, c_body, jnp.int32(0), unroll=4)

    sid0 = sid_ref[pl.ds(0, L)]
    sid1 = sid_ref[pl.ds(L, L)]
    tv_hi = jnp.full((L,), NEG_INF, jnp.float32)
    tv_lo = jnp.full((L,), NEG_INF, jnp.float32)
    ti_hi = jnp.zeros((L,), jnp.int32)
    ti_lo = jnp.zeros((L,), jnp.int32)
    for j in range(2 * L):
        sid = sid0[j] if j < L else sid1[j - L]
        base = (sid >> 4) * (GROUP_VREGS * L) + (sid & (L - 1))
        idx = base + L * lane
        v = plsc.load_gather(buf, [idx])
        sv, si = _sort_kv_desc(v, idx)
        tv_hi, ti_hi, tv_lo, ti_lo = _merge16_kv(
            tv_hi, ti_hi, tv_lo, ti_lo, sv, si)

    plsc.store_scatter(out_ref, [prev_i[pl.ds(0, L)]], zero16)
    plsc.store_scatter(out_ref, [prev_i[pl.ds(L, L)]], zero16)
    plsc.store_scatter(out_ref, [ti_hi], tv_hi)
    plsc.store_scatter(out_ref, [ti_lo], tv_lo)
    prev_i[pl.ds(0, L)] = ti_hi
    prev_i[pl.ds(L, L)] = ti_lo


def _sc_topk_scatter(latents):
    mesh = plsc.VectorSubcoreMesh(core_axis_name="c", subcore_axis_name="s")

    @functools.partial(
        pl.kernel,
        out_type=jax.ShapeDtypeStruct((N_TOK, D_SAE), jnp.float32),
        mesh=mesh,
        compiler_params=pltpu.CompilerParams(needs_layout_passes=False),
        scratch_types=[
            pltpu.VMEM((D_SAE,), jnp.float32),
            pltpu.VMEM((D_SAE,), jnp.float32),
            pltpu.VMEM((D_SAE,), jnp.float32),
            pltpu.VMEM((N_GROUPS * L,), jnp.float32),
            pltpu.VMEM((D_SAE // L + L,), jnp.int32),
            pltpu.VMEM((2 * L,), jnp.int32),
            pltpu.SemaphoreType.DMA,
            pltpu.SemaphoreType.DMA,
            pltpu.SemaphoreType.DMA,
        ],
    )
    def sc_kernel(lat_hbm, out_hbm, inbuf0, inbuf1, outbuf, qmax_ref, sid_ref,
                  prev_i, sem0, sem1, osem):
        wid = lax.axis_index("s") * 2 + lax.axis_index("c")
        row0 = wid * ROWS_PER_W
        lane = jnp.arange(L, dtype=jnp.int32)

        def z_body(i, _):
            outbuf[pl.ds(i * L, L)] = jnp.zeros((L,), jnp.float32)
            return 0
        lax.fori_loop(0, D_SAE // L, z_body, 0)
        prev_i[pl.ds(0, L)] = lane
        prev_i[pl.ds(L, L)] = lane + L

        in_sems = (sem0, sem1)
        in_bufs = (inbuf0, inbuf1)
        pltpu.async_copy(lat_hbm.at[row0], inbuf0, sem0)

        def pair_body(p, _):
            for phase in range(2):
                r = p * 2 + phase
                b = phase
                @pl.when(r + 1 < ROWS_PER_W)
                def _():
                    pltpu.async_copy(lat_hbm.at[row0 + r + 1],
                                     in_bufs[1 - b], in_sems[1 - b])
                pltpu.make_async_copy(lat_hbm.at[row0 + r], in_bufs[b],
                                      in_sems[b]).wait()
                @pl.when(r > 0)
                def _():
                    pltpu.make_async_copy(outbuf, out_hbm.at[row0 + r - 1],
                                          osem).wait()
                _row_topk_scatter(in_bufs[b], qmax_ref, sid_ref, outbuf,
                                  prev_i)
                pltpu.async_copy(outbuf, out_hbm.at[row0 + r], osem)
            return 0
        lax.fori_loop(0, ROWS_PER_W // 2, pair_body, 0)
        pltpu.make_async_copy(outbuf, out_hbm.at[row0 + ROWS_PER_W - 1],
                              osem).wait()

    return sc_kernel(latents)


def kernel(x, W_enc, b_enc, W_dec, b_dec):
    latents = _encoder(x, W_enc, b_enc)
    sparse_latents = _sc_topk_scatter(latents)
    recon = _decoder(sparse_latents, W_dec.astype(jnp.bfloat16), b_dec)
    return (recon, sparse_latents, latents)


# speedup vs baseline: 10.1007x; 1.0846x over previous
import functools

import jax
import jax.numpy as jnp
from jax import lax
from jax.experimental import pallas as pl
from jax.experimental.pallas import tpu as pltpu
from jax.experimental.pallas import tpu_sc as plsc

D_MODEL = 1024
D_SAE = 16384
N_TOK = 8192
TOPK = 32

NEG_INF = float("-inf")
L = 16
N_WORKERS = 32
ROWS_PER_W = N_TOK // N_WORKERS
N_GROUPS = 64
GROUP_VREGS = D_SAE // N_GROUPS // L
CAP_ELEMS = 2048


BT_ENC = 512
BD_ENC = 2048


def _enc_body(x_ref, w_ref, b_ref, out_ref):
    acc = jnp.dot(x_ref[...], w_ref[...], preferred_element_type=jnp.float32)
    out_ref[...] = acc + b_ref[...]


def _encoder(x, W_enc, b_enc):
    grid = (D_SAE // BD_ENC, N_TOK // BT_ENC)
    return pl.pallas_call(
        _enc_body,
        grid=grid,
        in_specs=[
            pl.BlockSpec((BT_ENC, D_MODEL), lambda j, i: (i, 0)),
            pl.BlockSpec((D_MODEL, BD_ENC), lambda j, i: (0, j)),
            pl.BlockSpec((1, BD_ENC), lambda j, i: (0, j)),
        ],
        out_specs=pl.BlockSpec((BT_ENC, BD_ENC), lambda j, i: (i, j)),
        out_shape=jax.ShapeDtypeStruct((N_TOK, D_SAE), jnp.float32),
    )(x, W_enc, b_enc.reshape(1, D_SAE))


BT_DEC = 128


def _dec_body(s_ref, w_ref, b_ref, out_ref):
    s = s_ref[...].astype(jnp.bfloat16)
    acc = jnp.dot(s, w_ref[...], preferred_element_type=jnp.float32)
    out_ref[...] = acc + b_ref[...]


def _decoder(sparse, W_dec_bf16, b_dec):
    grid = (N_TOK // BT_DEC,)
    return pl.pallas_call(
        _dec_body,
        grid=grid,
        in_specs=[
            pl.BlockSpec((BT_DEC, D_SAE), lambda i: (i, 0)),
            pl.BlockSpec((D_SAE, D_MODEL), lambda i: (0, 0)),
            pl.BlockSpec((1, D_MODEL), lambda i: (0, 0)),
        ],
        out_specs=pl.BlockSpec((BT_DEC, D_MODEL), lambda i: (i, 0)),
        out_shape=jax.ShapeDtypeStruct((N_TOK, D_MODEL), jnp.float32),
    )(sparse, W_dec_bf16, b_dec.reshape(1, D_MODEL))


def _any(mask):
    return plsc.all_reduce_population_count(mask)[0] > 0


def _sort_desc(v):
    s, _ = plsc.sort_key_val(v, v, descending=True)
    return s


def _merge16_vals(t_hi, t_lo, s_desc):
    x = jnp.maximum(t_lo, lax.rev(s_desc, (0,)))
    xs = _sort_desc(x)
    rxs = lax.rev(xs, (0,))
    hi = jnp.maximum(t_hi, rxs)
    lo = jnp.minimum(t_hi, rxs)
    return _sort_desc(hi), _sort_desc(lo)


def _sort_kv_desc(v, i):
    return plsc.sort_key_val(v, i, descending=True)


def _merge16_kv(tv_hi, ti_hi, tv_lo, ti_lo, sv_desc, si_desc):
    rv = lax.rev(sv_desc, (0,))
    ri = lax.rev(si_desc, (0,))
    c = tv_lo >= rv
    xv = jnp.where(c, tv_lo, rv)
    xi = jnp.where(c, ti_lo, ri)
    xv, xi = _sort_kv_desc(xv, xi)
    rxv = lax.rev(xv, (0,))
    rxi = lax.rev(xi, (0,))
    c2 = tv_hi >= rxv
    hv = jnp.where(c2, tv_hi, rxv)
    hi_ = jnp.where(c2, ti_hi, rxi)
    lv = jnp.where(c2, rxv, tv_hi)
    li = jnp.where(c2, rxi, ti_hi)
    tv_hi, ti_hi = _sort_kv_desc(hv, hi_)
    tv_lo, ti_lo = _sort_kv_desc(lv, li)
    return tv_hi, ti_hi, tv_lo, ti_lo


def _kv_tree(lo, hi, leaf_fn):
    n = hi - lo
    if n == 2:
        av, ai = leaf_fn(lo)
        bv, bi = leaf_fn(lo + 1)
        neg = jnp.full((L,), NEG_INF, jnp.float32)
        zi = jnp.zeros((L,), jnp.int32)
        return _merge16_kv(av, ai, neg, zi, bv, bi)
    mid = lo + n // 2
    left = _kv_tree(lo, mid, leaf_fn)
    right = _kv_tree(mid, hi, leaf_fn)
    out = _merge16_kv(*left, right[0], right[1])
    return _merge16_kv(*out, right[2], right[3])


def _row_topk_scatter(buf, rowbase, qmax_ref, out_ref, prev_i):
    lane = jnp.arange(L, dtype=jnp.int32)
    zero16 = jnp.zeros((L,), jnp.float32)

    def a_body(g, _):
        base = rowbase + g * (GROUP_VREGS * L)
        m = buf[pl.ds(base, L)]
        for i in range(1, GROUP_VREGS):
            m = jnp.maximum(m, buf[pl.ds(base + i * L, L)])
        qmax_ref[pl.ds(g * L, L)] = m
        return 0
    lax.fori_loop(0, N_GROUPS, a_body, 0, unroll=4)

    def b_leaf(g):
        return _sort_kv_desc(qmax_ref[pl.ds(g * L, L)], lane + g * L)
    _, sid_hi, _, sid_lo = _kv_tree(0, N_GROUPS, b_leaf)

    def c_leaf(j):
        sid = sid_hi[j] if j < L else sid_lo[j - L]
        base = (sid >> 4) * (GROUP_VREGS * L) + (sid & (L - 1))
        idx = base + L * lane
        v = plsc.load_gather(buf, [rowbase + idx])
        sv, si = _sort_kv_desc(v, idx)
        return sv, si
    tv_hi, ti_hi, tv_lo, ti_lo = _kv_tree(0, 2 * L, c_leaf)

    plsc.store_scatter(out_ref, [prev_i[pl.ds(0, L)]], zero16)
    plsc.store_scatter(out_ref, [prev_i[pl.ds(L, L)]], zero16)
    plsc.store_scatter(out_ref, [ti_hi], tv_hi)
    plsc.store_scatter(out_ref, [ti_lo], tv_lo)
    prev_i[pl.ds(0, L)] = ti_hi
    prev_i[pl.ds(L, L)] = ti_lo


def _sc_topk_scatter(latents):
    mesh = plsc.VectorSubcoreMesh(core_axis_name="c", subcore_axis_name="s")

    @functools.partial(
        pl.kernel,
        out_type=jax.ShapeDtypeStruct((N_TOK, D_SAE), jnp.float32),
        mesh=mesh,
        compiler_params=pltpu.CompilerParams(needs_layout_passes=False),
        scratch_types=[
            pltpu.VMEM((2 * D_SAE,), jnp.float32),
            pltpu.VMEM((D_SAE,), jnp.float32),
            pltpu.VMEM((N_GROUPS * L,), jnp.float32),
            pltpu.VMEM((2 * L,), jnp.int32),
            pltpu.SemaphoreType.DMA,
            pltpu.SemaphoreType.DMA,
            pltpu.SemaphoreType.DMA,
        ],
    )
    def sc_kernel(lat_hbm, out_hbm, inbuf, outbuf, qmax_ref, prev_i,
                  sem0, sem1, osem):
        wid = lax.axis_index("s") * 2 + lax.axis_index("c")
        row0 = wid * ROWS_PER_W
        lane = jnp.arange(L, dtype=jnp.int32)

        def z_body(i, _):
            outbuf[pl.ds(i * L, L)] = jnp.zeros((L,), jnp.float32)
            return 0
        lax.fori_loop(0, D_SAE // L, z_body, 0)
        prev_i[pl.ds(0, L)] = lane
        prev_i[pl.ds(L, L)] = lane + L

        half0 = inbuf.at[pl.ds(0, D_SAE)]
        half1 = inbuf.at[pl.ds(D_SAE, D_SAE)]
        pltpu.async_copy(lat_hbm.at[row0], half0, sem0)

        def row_body(r, _):
            par = r & 1
            @pl.when((r + 1 < ROWS_PER_W) & (par == 0))
            def _():
                pltpu.async_copy(lat_hbm.at[row0 + r + 1], half1, sem1)

            @pl.when((r + 1 < ROWS_PER_W) & (par == 1))
            def _():
                pltpu.async_copy(lat_hbm.at[row0 + r + 1], half0, sem0)

            @pl.when(par == 0)
            def _():
                pltpu.make_async_copy(lat_hbm.at[row0 + r], half0, sem0).wait()

            @pl.when(par == 1)
            def _():
                pltpu.make_async_copy(lat_hbm.at[row0 + r], half1, sem1).wait()

            @pl.when(r > 0)
            def _():
                pltpu.make_async_copy(outbuf, out_hbm.at[row0 + r - 1],
                                      osem).wait()
            _row_topk_scatter(inbuf, par * D_SAE, qmax_ref, outbuf, prev_i)
            pltpu.async_copy(outbuf, out_hbm.at[row0 + r], osem)
            return 0
        lax.fori_loop(0, ROWS_PER_W, row_body, 0)
        pltpu.make_async_copy(outbuf, out_hbm.at[row0 + ROWS_PER_W - 1],
                              osem).wait()

    return sc_kernel(latents)


def kernel(x, W_enc, b_enc, W_dec, b_dec):
    latents = _encoder(x, W_enc, b_enc)
    sparse_latents = _sc_topk_scatter(latents)
    recon = _decoder(sparse_latents, W_dec.astype(jnp.bfloat16), b_dec)
    return (recon, sparse_latents, latents)


# speedup vs baseline: 15.3458x; 1.5193x over previous
import functools

import jax
import jax.numpy as jnp
from jax import lax
from jax.experimental import pallas as pl
from jax.experimental.pallas import tpu as pltpu
from jax.experimental.pallas import tpu_sc as plsc

D_MODEL = 1024
D_SAE = 16384
N_TOK = 8192
TOPK = 32

NEG_INF = float("-inf")
L = 16
N_WORKERS = 32
ROWS_PER_W = N_TOK // N_WORKERS
N_GROUPS = 64
GROUP_VREGS = D_SAE // N_GROUPS // L
CAP_ELEMS = 2048


BT_ENC = 512
BD_ENC = 2048


def _enc_body(x_ref, w_ref, b_ref, out_ref):
    acc = jnp.dot(x_ref[...], w_ref[...], preferred_element_type=jnp.float32)
    out_ref[...] = acc + b_ref[...]


def _encoder(x, W_enc, b_enc):
    grid = (D_SAE // BD_ENC, N_TOK // BT_ENC)
    return pl.pallas_call(
        _enc_body,
        grid=grid,
        in_specs=[
            pl.BlockSpec((BT_ENC, D_MODEL), lambda j, i: (i, 0)),
            pl.BlockSpec((D_MODEL, BD_ENC), lambda j, i: (0, j)),
            pl.BlockSpec((1, BD_ENC), lambda j, i: (0, j)),
        ],
        out_specs=pl.BlockSpec((BT_ENC, BD_ENC), lambda j, i: (i, j)),
        out_shape=jax.ShapeDtypeStruct((N_TOK, D_SAE), jnp.float32),
    )(x, W_enc, b_enc.reshape(1, D_SAE))


BT_DEC = 128


def _dec_body(s_ref, w_ref, b_ref, out_ref):
    s = s_ref[...].astype(jnp.bfloat16)
    acc = jnp.dot(s, w_ref[...], preferred_element_type=jnp.float32)
    out_ref[...] = acc + b_ref[...]


def _decoder(sparse, W_dec_bf16, b_dec):
    grid = (N_TOK // BT_DEC,)
    return pl.pallas_call(
        _dec_body,
        grid=grid,
        in_specs=[
            pl.BlockSpec((BT_DEC, D_SAE), lambda i: (i, 0)),
            pl.BlockSpec((D_SAE, D_MODEL), lambda i: (0, 0)),
            pl.BlockSpec((1, D_MODEL), lambda i: (0, 0)),
        ],
        out_specs=pl.BlockSpec((BT_DEC, D_MODEL), lambda i: (i, 0)),
        out_shape=jax.ShapeDtypeStruct((N_TOK, D_MODEL), jnp.float32),
    )(sparse, W_dec_bf16, b_dec.reshape(1, D_MODEL))


def _sort_kv(v, i, desc):
    return plsc.sort_key_val(v, i, descending=desc)


def _bf_max(av, ai, bv, bi):
    c = av >= bv
    return jnp.maximum(av, bv), jnp.where(c, ai, bi)


def _bf_min(av, ai, bv, bi):
    c = av >= bv
    return jnp.minimum(av, bv), jnp.where(c, bi, ai)


def _arrange32(c0v, c0i, c1v, c1i, desc):
    ev, ei = _bf_max(c0v, c0i, c1v, c1i)
    fv, fi = _bf_min(c0v, c0i, c1v, c1i)
    if desc:
        h = _sort_kv(ev, ei, True)
        l = _sort_kv(fv, fi, True)
        return h[0], h[1], l[0], l[1]
    h = _sort_kv(fv, fi, False)
    l = _sort_kv(ev, ei, False)
    return h[0], h[1], l[0], l[1]


def _kv_tree(lo, hi, leaf_fn, desc=True):
    n = hi - lo
    if n == 2:
        av, ai = leaf_fn(lo, desc)
        bv, bi = leaf_fn(lo + 1, not desc)
        return _arrange32(*_bf_max(av, ai, bv, bi),
                          *_bf_min(av, ai, bv, bi), desc=desc)
    mid = lo + n // 2
    a = _kv_tree(lo, mid, leaf_fn, desc)
    b = _kv_tree(mid, hi, leaf_fn, not desc)
    c0 = _bf_max(a[0], a[1], b[0], b[1])
    c1 = _bf_max(a[2], a[3], b[2], b[3])
    return _arrange32(*c0, *c1, desc=desc)


def _row_topk_scatter(buf, rowbase, qmax_ref, out_ref, prev_i):
    lane = jnp.arange(L, dtype=jnp.int32)
    zero16 = jnp.zeros((L,), jnp.float32)

    def a_body(g, _):
        base = rowbase + g * (GROUP_VREGS * L)
        m = buf[pl.ds(base, L)]
        for i in range(1, GROUP_VREGS):
            m = jnp.maximum(m, buf[pl.ds(base + i * L, L)])
        qmax_ref[pl.ds(g * L, L)] = m
        return 0
    lax.fori_loop(0, N_GROUPS, a_body, 0, unroll=4)

    def b_leaf(g, d):
        return _sort_kv(qmax_ref[pl.ds(g * L, L)], lane + g * L, d)
    _, sid_hi, _, sid_lo = _kv_tree(0, N_GROUPS, b_leaf)

    def c_leaf(j, d):
        sid = sid_hi[j] if j < L else sid_lo[j - L]
        base = (sid >> 4) * (GROUP_VREGS * L) + (sid & (L - 1))
        idx = base + L * lane
        v = plsc.load_gather(buf, [rowbase + idx])
        return _sort_kv(v, idx, d)
    tv_hi, ti_hi, tv_lo, ti_lo = _kv_tree(0, 2 * L, c_leaf)

    plsc.store_scatter(out_ref, [prev_i[pl.ds(0, L)]], zero16)
    plsc.store_scatter(out_ref, [prev_i[pl.ds(L, L)]], zero16)
    plsc.store_scatter(out_ref, [ti_hi], tv_hi)
    plsc.store_scatter(out_ref, [ti_lo], tv_lo)
    prev_i[pl.ds(0, L)] = ti_hi
    prev_i[pl.ds(L, L)] = ti_lo


def _sc_topk_scatter(latents):
    mesh = plsc.VectorSubcoreMesh(core_axis_name="c", subcore_axis_name="s")

    @functools.partial(
        pl.kernel,
        out_type=jax.ShapeDtypeStruct((N_TOK, D_SAE), jnp.float32),
        mesh=mesh,
        compiler_params=pltpu.CompilerParams(needs_layout_passes=False),
        scratch_types=[
            pltpu.VMEM((2 * D_SAE,), jnp.float32),
            pltpu.VMEM((D_SAE,), jnp.float32),
            pltpu.VMEM((N_GROUPS * L,), jnp.float32),
            pltpu.VMEM((2 * L,), jnp.int32),
            pltpu.SemaphoreType.DMA,
            pltpu.SemaphoreType.DMA,
            pltpu.SemaphoreType.DMA,
        ],
    )
    def sc_kernel(lat_hbm, out_hbm, inbuf, outbuf, qmax_ref, prev_i,
                  sem0, sem1, osem):
        wid = lax.axis_index("s") * 2 + lax.axis_index("c")
        row0 = wid * ROWS_PER_W
        lane = jnp.arange(L, dtype=jnp.int32)

        def z_body(i, _):
            outbuf[pl.ds(i * L, L)] = jnp.zeros((L,), jnp.float32)
            return 0
        lax.fori_loop(0, D_SAE // L, z_body, 0)
        prev_i[pl.ds(0, L)] = lane
        prev_i[pl.ds(L, L)] = lane + L

        half0 = inbuf.at[pl.ds(0, D_SAE)]
        half1 = inbuf.at[pl.ds(D_SAE, D_SAE)]
        pltpu.async_copy(lat_hbm.at[row0], half0, sem0)

        def row_body(r, _):
            par = r & 1
            @pl.when((r + 1 < ROWS_PER_W) & (par == 0))
            def _():
                pltpu.async_copy(lat_hbm.at[row0 + r + 1], half1, sem1)

            @pl.when((r + 1 < ROWS_PER_W) & (par == 1))
            def _():
                pltpu.async_copy(lat_hbm.at[row0 + r + 1], half0, sem0)

            @pl.when(par == 0)
            def _():
                pltpu.make_async_copy(lat_hbm.at[row0 + r], half0, sem0).wait()

            @pl.when(par == 1)
            def _():
                pltpu.make_async_copy(lat_hbm.at[row0 + r], half1, sem1).wait()

            @pl.when(r > 0)
            def _():
                pltpu.make_async_copy(outbuf, out_hbm.at[row0 + r - 1],
                                      osem).wait()
            _row_topk_scatter(inbuf, par * D_SAE, qmax_ref, outbuf, prev_i)
            pltpu.async_copy(outbuf, out_hbm.at[row0 + r], osem)
            return 0
        lax.fori_loop(0, ROWS_PER_W, row_body, 0)
        pltpu.make_async_copy(outbuf, out_hbm.at[row0 + ROWS_PER_W - 1],
                              osem).wait()

    return sc_kernel(latents)


def kernel(x, W_enc, b_enc, W_dec, b_dec):
    latents = _encoder(x, W_enc, b_enc)
    sparse_latents = _sc_topk_scatter(latents)
    recon = _decoder(sparse_latents, W_dec.astype(jnp.bfloat16), b_dec)
    return (recon, sparse_latents, latents)
